# baseline (speedup 1.0000x reference)
import functools

import jax
import jax.numpy as jnp
from jax import lax
from jax.experimental import pallas as pl
from jax.experimental.pallas import tpu as pltpu
from jax.experimental.pallas import tpu_sc as plsc

_START, _END = 512, 2560
_ZROWS = 64


def _sc_body(g_hbm, o_hbm, zbuf):
    nbatch, g_rows, feat = g_hbm.shape
    seq = o_hbm.shape[1]
    nc = 2
    wid = lax.axis_index("s") * nc + lax.axis_index("c")
    chunks_per_batch = seq // 512
    b = wid // chunks_per_batch
    k = wid % chunks_per_batch
    lo = _START // 512
    hi = _END // 512
    inside = (k >= lo) & (k < hi)

    @pl.when(inside)
    def _copy():
        pltpu.sync_copy(
            g_hbm.at[b, pl.ds((k - lo) * 512, 512)],
            o_hbm.at[b, pl.ds(k * 512, 512)],
        )

    @pl.when(jnp.logical_not(inside))
    def _zero():
        def zrow(i, carry):
            def zcol(j, c):
                zbuf[i, pl.ds(j * 16, 16)] = jnp.zeros((16,), jnp.float32)
                return c

            return lax.fori_loop(0, feat // 16, zcol, carry)

        lax.fori_loop(0, _ZROWS, zrow, 0)

        def zdma(i, carry):
            pltpu.sync_copy(
                zbuf, o_hbm.at[b, pl.ds(k * 512 + i * _ZROWS, _ZROWS)]
            )
            return carry

        lax.fori_loop(0, 512 // _ZROWS, zdma, 0)


def kernel(grad_last, input):
    b0, b1, g_rows, feat = grad_last.shape
    seq = input.shape[1]
    nb = b0 * b1
    g = grad_last.reshape(nb, g_rows, feat)

    mesh = plsc.VectorSubcoreMesh(core_axis_name="c", subcore_axis_name="s")
    run = pl.kernel(
        _sc_body,
        out_type=jax.ShapeDtypeStruct((nb, seq, feat), grad_last.dtype),
        mesh=mesh,
        scratch_types=[pltpu.VMEM((_ZROWS, feat), jnp.float32)],
    )
    out = run(g)
    return out.reshape(b0, b1, seq, feat)


# speedup vs baseline: 18.6124x; 18.6124x over previous
import jax
import jax.numpy as jnp
from jax import lax
from jax.experimental import pallas as pl
from jax.experimental.pallas import tpu as pltpu
from jax.experimental.pallas import tpu_sc as plsc

_START, _END = 512, 2560
_CHUNK = 32


def _sc_body(g_hbm, o_hbm, vb0, vb1, zb, si0, si1, so0, so1, sz):
    nbatch, g_rows, feat = g_hbm.shape
    seq = o_hbm.shape[1]
    nc = 2
    wid = lax.axis_index("s") * nc + lax.axis_index("c")
    nw = 32
    w_rows = nbatch * g_rows // nw
    per_b = g_rows // w_rows
    b = wid // per_b
    g_off = (wid % per_b) * w_rows
    o_off = _START + g_off
    z_rows = w_rows
    z_off_raw = (wid % per_b) * z_rows
    z_off = jnp.where(
        z_off_raw < _START, z_off_raw, _END + (z_off_raw - _START)
    )

    def zrow(i, c):
        for j in range(feat // 16):
            zb[i, pl.ds(j * 16, 16)] = jnp.zeros((16,), jnp.float32)
        return c

    lax.fori_loop(0, _CHUNK, zrow, 0)

    n_z = z_rows // _CHUNK
    zdmas = [
        pltpu.make_async_copy(
            zb, o_hbm.at[b, pl.ds(z_off + c * _CHUNK, _CHUNK)], sz
        )
        for c in range(n_z)
    ]
    for d in zdmas:
        d.start()

    n_c = w_rows // _CHUNK
    bufs = (vb0, vb1)
    isems = (si0, si1)
    osems = (so0, so1)
    cin = [
        pltpu.make_async_copy(
            g_hbm.at[b, pl.ds(g_off + c * _CHUNK, _CHUNK)],
            bufs[c % 2],
            isems[c % 2],
        )
        for c in range(n_c)
    ]
    cout = [
        pltpu.make_async_copy(
            bufs[c % 2],
            o_hbm.at[b, pl.ds(o_off + c * _CHUNK, _CHUNK)],
            osems[c % 2],
        )
        for c in range(n_c)
    ]
    cin[0].start()
    for c in range(n_c):
        if c + 1 < n_c:
            if c >= 1:
                cout[c - 1].wait()
            cin[c + 1].start()
        cin[c].wait()
        cout[c].start()
    cout[n_c - 2].wait()
    cout[n_c - 1].wait()
    for d in zdmas:
        d.wait()


def kernel(grad_last, input):
    b0, b1, g_rows, feat = grad_last.shape
    seq = input.shape[1]
    nb = b0 * b1
    g = grad_last.reshape(nb, g_rows, feat)

    mesh = plsc.VectorSubcoreMesh(core_axis_name="c", subcore_axis_name="s")
    run = pl.kernel(
        _sc_body,
        out_type=jax.ShapeDtypeStruct((nb, seq, feat), grad_last.dtype),
        mesh=mesh,
        scratch_types=[
            pltpu.VMEM((_CHUNK, feat), jnp.float32),
            pltpu.VMEM((_CHUNK, feat), jnp.float32),
            pltpu.VMEM((_CHUNK, feat), jnp.float32),
            pltpu.SemaphoreType.DMA,
            pltpu.SemaphoreType.DMA,
            pltpu.SemaphoreType.DMA,
            pltpu.SemaphoreType.DMA,
            pltpu.SemaphoreType.DMA,
        ],
    )
    out = run(g)
    return out.reshape(b0, b1, seq, feat)


# speedup vs baseline: 19.3296x; 1.0385x over previous
import jax
import jax.numpy as jnp
from jax import lax
from jax.experimental import pallas as pl
from jax.experimental.pallas import tpu as pltpu
from jax.experimental.pallas import tpu_sc as plsc

_START, _END = 512, 2560
_CHUNK = 32


def _sc_body(g_hbm, o_hbm, vb0, vb1, zb, si0, si1, so0, so1, sz):
    nbatch, g_rows, feat = g_hbm.shape
    seq = o_hbm.shape[1]
    nc = 2
    wid = lax.axis_index("s") * nc + lax.axis_index("c")
    nw = 32
    w_rows = nbatch * g_rows // nw
    per_b = g_rows // w_rows
    b = wid // per_b
    g_off = (wid % per_b) * w_rows
    o_off = _START + g_off
    z_rows = w_rows
    z_off_raw = (wid % per_b) * z_rows
    z_off = jnp.where(
        z_off_raw < _START, z_off_raw, _END + (z_off_raw - _START)
    )

    n_c = w_rows // _CHUNK
    bufs = (vb0, vb1)
    isems = (si0, si1)
    osems = (so0, so1)
    cin = [
        pltpu.make_async_copy(
            g_hbm.at[b, pl.ds(g_off + c * _CHUNK, _CHUNK)],
            bufs[c % 2],
            isems[c % 2],
        )
        for c in range(n_c)
    ]
    cout = [
        pltpu.make_async_copy(
            bufs[c % 2],
            o_hbm.at[b, pl.ds(o_off + c * _CHUNK, _CHUNK)],
            osems[c % 2],
        )
        for c in range(n_c)
    ]
    cin[0].start()

    def zrow(i, c):
        for j in range(feat // 16):
            zb[i, pl.ds(j * 16, 16)] = jnp.zeros((16,), jnp.float32)
        return c

    lax.fori_loop(0, _CHUNK, zrow, 0)

    n_z = z_rows // _CHUNK
    zdmas = [
        pltpu.make_async_copy(
            zb, o_hbm.at[b, pl.ds(z_off + c * _CHUNK, _CHUNK)], sz
        )
        for c in range(n_z)
    ]
    for d in zdmas:
        d.start()

    for c in range(n_c):
        if c + 1 < n_c:
            if c >= 1:
                cout[c - 1].wait()
            cin[c + 1].start()
        cin[c].wait()
        cout[c].start()
    cout[n_c - 2].wait()
    cout[n_c - 1].wait()
    for d in zdmas:
        d.wait()


def kernel(grad_last, input):
    b0, b1, g_rows, feat = grad_last.shape
    seq = input.shape[1]
    nb = b0 * b1
    g = grad_last.reshape(nb, g_rows, feat)

    mesh = plsc.VectorSubcoreMesh(core_axis_name="c", subcore_axis_name="s")
    run = pl.kernel(
        _sc_body,
        out_type=jax.ShapeDtypeStruct((nb, seq, feat), grad_last.dtype),
        mesh=mesh,
        scratch_types=[
            pltpu.VMEM((_CHUNK, feat), jnp.float32),
            pltpu.VMEM((_CHUNK, feat), jnp.float32),
            pltpu.VMEM((_CHUNK, feat), jnp.float32),
            pltpu.SemaphoreType.DMA,
            pltpu.SemaphoreType.DMA,
            pltpu.SemaphoreType.DMA,
            pltpu.SemaphoreType.DMA,
            pltpu.SemaphoreType.DMA,
        ],
    )
    out = run(g)
    return out.reshape(b0, b1, seq, feat)
